# Initial kernel scaffold; baseline (speedup 1.0000x reference)
import functools

import jax
import jax.numpy as jnp
from jax import lax
from jax.experimental import pallas as pl
from jax.experimental.pallas import tpu as pltpu
from jax.experimental.pallas import tpu_sc as plsc

N = 10000
E = 320000
D = 128
H = 64
NC = 2
NS = 16
L = 16
C = 80
CHUNKS_PER_TILE = E // (NS * C)
ROWS_PER_TILE = N // NS


def _sc_body(x2, col2, row2, ev2, out, part, gbuf, colb, rowb, evb, gidx, sem):
    c = lax.axis_index("c")
    s = lax.axis_index("s")
    base = s * CHUNKS_PER_TILE

    pltpu.sync_copy(col2.at[pl.ds(base, CHUNKS_PER_TILE)], colb)
    pltpu.sync_copy(row2.at[pl.ds(base, CHUNKS_PER_TILE)], rowb)
    pltpu.sync_copy(ev2.at[pl.ds(base, CHUNKS_PER_TILE)], evb)

    zeros = jnp.zeros((L,), jnp.float32)
    def zrow(i, _):
        for k in range(H // L):
            gbuf[i, pl.ds(k * L, L)] = zeros
        return 0
    lax.fori_loop(0, C, zrow, 0)
    row0 = s * ROWS_PER_TILE
    nfull = ROWS_PER_TILE // C
    tail = ROWS_PER_TILE - nfull * C
    for k in range(nfull):
        pltpu.sync_copy(gbuf, part.at[pl.ds(row0 + k * C, C)])
    pltpu.sync_copy(gbuf.at[pl.ds(0, tail)], part.at[pl.ds(row0 + nfull * C, tail)])
    plsc.subcore_barrier()

    def chunk_body(g, _):
        for k in range(C // L):
            v = colb[g, pl.ds(k * L, L)]
            gidx[pl.ds(k * L, L)] = v * 2 + c
        pltpu.async_copy(x2.at[gidx], gbuf, sem).wait()

        def edge_body(j, _):
            evs = plsc.load_gather(
                evb, [jnp.full((L,), g, jnp.int32), jnp.full((L,), j, jnp.int32)])
            for k in range(H // L):
                gbuf[j, pl.ds(k * L, L)] = gbuf[j, pl.ds(k * L, L)] * evs
            return 0
        lax.fori_loop(0, C, edge_body, 0)

        pltpu.sync_copy(gbuf, part.at[rowb.at[g]], add=True)
        return 0

    lax.fori_loop(0, CHUNKS_PER_TILE, chunk_body, 0)
    plsc.subcore_barrier()

    for k in range(nfull):
        pltpu.sync_copy(part.at[pl.ds(row0 + k * C, C)],
                        out.at[c, pl.ds(row0 + k * C, C)])
    pltpu.sync_copy(part.at[pl.ds(row0 + nfull * C, tail)],
                    out.at[c, pl.ds(row0 + nfull * C, tail)])


_sc_spmm = functools.partial(
    pl.kernel,
    out_type=jax.ShapeDtypeStruct((NC, N, H), jnp.float32),
    mesh=plsc.VectorSubcoreMesh(core_axis_name="c", subcore_axis_name="s"),
    scratch_types=[
        pltpu.VMEM_SHARED((N, H), jnp.float32),
        pltpu.VMEM((C, H), jnp.float32),
        pltpu.VMEM((CHUNKS_PER_TILE, C), jnp.int32),
        pltpu.VMEM((CHUNKS_PER_TILE, C), jnp.int32),
        pltpu.VMEM((CHUNKS_PER_TILE, C), jnp.float32),
        pltpu.VMEM((C,), jnp.int32),
        pltpu.SemaphoreType.DMA,
    ],
)(_sc_body)


def _tc_body(plo_ref, phi_ref, w0_ref, w1_ref, o_ref):
    acc = jnp.dot(plo_ref[...], w0_ref[...], preferred_element_type=jnp.float32)
    acc = acc + jnp.dot(phi_ref[...], w1_ref[...],
                        preferred_element_type=jnp.float32)
    o_ref[...] = jnp.maximum(acc, 0.0)


_BM = 1000


def _tc_matmul(plo, phi, w0, w1):
    return pl.pallas_call(
        _tc_body,
        grid=(N // _BM,),
        in_specs=[
            pl.BlockSpec((_BM, H), lambda i: (i, 0)),
            pl.BlockSpec((_BM, H), lambda i: (i, 0)),
            pl.BlockSpec((H, D), lambda i: (0, 0)),
            pl.BlockSpec((H, D), lambda i: (0, 0)),
        ],
        out_specs=pl.BlockSpec((_BM, D), lambda i: (i, 0)),
        out_shape=jax.ShapeDtypeStruct((N, D), jnp.float32),
    )(plo, phi, w0, w1)


def kernel(x, W, w_comb, edge_vals, edge_index):
    x2 = x.reshape(2 * N, H)
    col2 = edge_index[1].reshape(E // C, C)
    row2 = edge_index[0].reshape(E // C, C)
    ev2 = edge_vals.reshape(E // C, C)
    parts = _sc_spmm(x2, col2, row2, ev2)
    Ws = W * w_comb[0, 0]
    return _tc_matmul(parts[0], parts[1], Ws[:H], Ws[H:])


# speedup vs baseline: 3.7187x; 3.7187x over previous
import functools

import jax
import jax.numpy as jnp
from jax import lax
from jax.experimental import pallas as pl
from jax.experimental.pallas import tpu as pltpu
from jax.experimental.pallas import tpu_sc as plsc

N = 10000
E = 320000
D = 128
H = 64
NC = 2
NS = 16
L = 16
C = 80
EP = E // NS
CHUNKS_PER_TILE = EP // C
ROWS_PER_TILE = N // NS


def _sc_body(x2, col1, row1, ev1, out, part, gbuf, colb, rowb, evb, gidx, ridx,
             sem):
    c = lax.axis_index("c")
    s = lax.axis_index("s")

    pltpu.sync_copy(col1.at[pl.ds(s * EP, EP)], colb)
    pltpu.sync_copy(row1.at[pl.ds(s * EP, EP)], rowb)
    pltpu.sync_copy(ev1.at[pl.ds(s * EP, EP)], evb)

    zeros = jnp.zeros((L,), jnp.float32)
    def zrow(i, _):
        for k in range(H // L):
            gbuf[i, pl.ds(k * L, L)] = zeros
        return 0
    lax.fori_loop(0, C, zrow, 0)
    row0 = s * ROWS_PER_TILE
    nfull = ROWS_PER_TILE // C
    tail = ROWS_PER_TILE - nfull * C
    for k in range(nfull):
        pltpu.sync_copy(gbuf, part.at[pl.ds(row0 + k * C, C)])
    pltpu.sync_copy(gbuf.at[pl.ds(0, tail)], part.at[pl.ds(row0 + nfull * C, tail)])
    plsc.subcore_barrier()

    def chunk_body(g, _):
        for k in range(C // L):
            v = colb[pl.ds(g * C + k * L, L)]
            gidx[pl.ds(k * L, L)] = v * 2 + c
            ridx[pl.ds(k * L, L)] = rowb[pl.ds(g * C + k * L, L)]
        pltpu.async_copy(x2.at[gidx], gbuf, sem).wait()

        def edge_body(j, _):
            evs = plsc.load_gather(evb, [jnp.full((L,), g * C + j, jnp.int32)])
            for k in range(H // L):
                gbuf[j, pl.ds(k * L, L)] = gbuf[j, pl.ds(k * L, L)] * evs
            return 0
        lax.fori_loop(0, C, edge_body, 0)

        pltpu.sync_copy(gbuf, part.at[ridx], add=True)
        return 0

    lax.fori_loop(0, CHUNKS_PER_TILE, chunk_body, 0)
    plsc.subcore_barrier()

    for k in range(nfull):
        pltpu.sync_copy(part.at[pl.ds(row0 + k * C, C)],
                        out.at[c, pl.ds(row0 + k * C, C)])
    pltpu.sync_copy(part.at[pl.ds(row0 + nfull * C, tail)],
                    out.at[c, pl.ds(row0 + nfull * C, tail)])


_sc_spmm = functools.partial(
    pl.kernel,
    out_type=jax.ShapeDtypeStruct((NC, N, H), jnp.float32),
    mesh=plsc.VectorSubcoreMesh(core_axis_name="c", subcore_axis_name="s"),
    compiler_params=pltpu.CompilerParams(use_tc_tiling_on_sc=False,
                                         needs_layout_passes=False),
    scratch_types=[
        pltpu.VMEM_SHARED((N, H), jnp.float32),
        pltpu.VMEM((C, H), jnp.float32),
        pltpu.VMEM((EP,), jnp.int32),
        pltpu.VMEM((EP,), jnp.int32),
        pltpu.VMEM((EP,), jnp.float32),
        pltpu.VMEM((C,), jnp.int32),
        pltpu.VMEM((C,), jnp.int32),
        pltpu.SemaphoreType.DMA,
    ],
)(_sc_body)


def _tc_body(plo_ref, phi_ref, w0_ref, w1_ref, o_ref):
    acc = jnp.dot(plo_ref[...], w0_ref[...], preferred_element_type=jnp.float32)
    acc = acc + jnp.dot(phi_ref[...], w1_ref[...],
                        preferred_element_type=jnp.float32)
    o_ref[...] = jnp.maximum(acc, 0.0)


_BM = 1000


def _tc_matmul(plo, phi, w0, w1):
    return pl.pallas_call(
        _tc_body,
        grid=(N // _BM,),
        in_specs=[
            pl.BlockSpec((_BM, H), lambda i: (i, 0)),
            pl.BlockSpec((_BM, H), lambda i: (i, 0)),
            pl.BlockSpec((H, D), lambda i: (0, 0)),
            pl.BlockSpec((H, D), lambda i: (0, 0)),
        ],
        out_specs=pl.BlockSpec((_BM, D), lambda i: (i, 0)),
        out_shape=jax.ShapeDtypeStruct((N, D), jnp.float32),
    )(plo, phi, w0, w1)


def kernel(x, W, w_comb, edge_vals, edge_index):
    x2 = x.reshape(2 * N, H)
    parts = _sc_spmm(x2, edge_index[1], edge_index[0], edge_vals)
    Ws = W * w_comb[0, 0]
    return _tc_matmul(parts[0], parts[1], Ws[:H], Ws[H:])
